# Initial kernel scaffold; baseline (speedup 1.0000x reference)
import jax
import jax.numpy as jnp
from jax.experimental import pallas as pl


def kernel(node_in_feat, edge_index, edge_id, norm, W_in, b_in, rel_W, W0, W_out, b_out):
    raise NotImplementedError("write your pallas kernel here")


# speedup vs baseline: 10.6462x; 10.6462x over previous
import functools

import jax
import jax.numpy as jnp
from jax import lax
from jax.experimental import pallas as pl
from jax.experimental.pallas import tpu as pltpu
from jax.experimental.pallas import tpu_sc as plsc

NC = 2
NS = 16
NW = NC * NS
LANES = 16
CH = 128


def _mm_bias_relu(x, w, b):
  M, K = x.shape
  Do = w.shape[1]
  BM = 1000
  nb = M // BM
  b2 = b.reshape(1, Do)

  def body(x_ref, w_ref, b_ref, o_ref):
    acc = jnp.dot(x_ref[...], w_ref[...], preferred_element_type=jnp.float32)
    o_ref[...] = jnp.maximum(acc + b_ref[...], 0.0)

  return pl.pallas_call(
      body,
      grid=(nb,),
      in_specs=[
          pl.BlockSpec((BM, K), lambda i: (i, 0)),
          pl.BlockSpec((K, Do), lambda i: (0, 0)),
          pl.BlockSpec((1, Do), lambda i: (0, 0)),
      ],
      out_specs=pl.BlockSpec((BM, Do), lambda i: (i, 0)),
      out_shape=jax.ShapeDtypeStruct((M, Do), jnp.float32),
  )(x, w, b2)


def _mm_rel(h, relw):
  M, K = h.shape
  R, _, Do = relw.shape
  BM = 1000
  nb = M // BM

  def body(x_ref, w_ref, o_ref):
    o_ref[...] = jnp.dot(x_ref[...], w_ref[0],
                         preferred_element_type=jnp.float32)

  return pl.pallas_call(
      body,
      grid=(R, nb),
      in_specs=[
          pl.BlockSpec((BM, K), lambda r, i: (i, 0)),
          pl.BlockSpec((1, K, Do), lambda r, i: (r, 0, 0)),
      ],
      out_specs=pl.BlockSpec((BM, Do), lambda r, i, _nb=nb: (r * _nb + i, 0)),
      out_shape=jax.ShapeDtypeStruct((R * M, Do), jnp.float32),
  )(h, relw)


def _mm_apply(h, w0, part):
  M, K = h.shape
  Do = w0.shape[1]
  BM = 1000
  nb = M // BM

  def body(x_ref, w_ref, p0_ref, p1_ref, o_ref):
    acc = jnp.dot(x_ref[...], w_ref[...], preferred_element_type=jnp.float32)
    o_ref[...] = jnp.maximum(acc + p0_ref[...] + p1_ref[...], 0.0)

  return pl.pallas_call(
      body,
      grid=(nb,),
      in_specs=[
          pl.BlockSpec((BM, K), lambda i: (i, 0)),
          pl.BlockSpec((K, Do), lambda i: (0, 0)),
          pl.BlockSpec((BM, Do), lambda i: (i, 0)),
          pl.BlockSpec((BM, Do), lambda i, _nb=nb: (_nb + i, 0)),
      ],
      out_specs=pl.BlockSpec((BM, Do), lambda i: (i, 0)),
      out_shape=jax.ShapeDtypeStruct((M, Do), jnp.float32),
  )(h, w0, part, part)


def _sc_edge_agg(table, src, eid, dst, norm, zeros_nd, n_nodes, d):
  ep = src.shape[0]
  n_chunks = ep // (NW * CH)
  epw = n_chunks * CH
  rows_per_sub = (n_nodes // NS) // 8 * 8
  tail_rows = n_nodes - NS * rows_per_sub

  mesh = plsc.VectorSubcoreMesh(core_axis_name="c", subcore_axis_name="s")

  @functools.partial(
      pl.kernel,
      out_type=jax.ShapeDtypeStruct((NC * n_nodes, d), jnp.float32),
      mesh=mesh,
      scratch_types=[
          pltpu.VMEM((CH,), jnp.int32),
          pltpu.VMEM((CH,), jnp.int32),
          pltpu.VMEM((CH,), jnp.int32),
          pltpu.VMEM((CH,), jnp.int32),
          pltpu.VMEM((CH,), jnp.float32),
          pltpu.VMEM((CH, d), jnp.float32),
          pltpu.VMEM_SHARED((n_nodes, d), jnp.float32),
          pltpu.SemaphoreType.DMA,
      ],
  )
  def k(table_h, src_h, eid_h, dst_h, norm_h, zeros_h, out_h,
        src_v, eid_v, idx_v, dst_v, norm_v, rows_v, acc, sem):
    cid = lax.axis_index("c")
    sid = lax.axis_index("s")
    wid = sid * NC + cid

    pltpu.sync_copy(zeros_h.at[pl.ds(sid * rows_per_sub, rows_per_sub)],
                    acc.at[pl.ds(sid * rows_per_sub, rows_per_sub)])
    if tail_rows:
      @pl.when(sid == 0)
      def _():
        pltpu.sync_copy(zeros_h.at[pl.ds(NS * rows_per_sub, tail_rows)],
                        acc.at[pl.ds(NS * rows_per_sub, tail_rows)])
    plsc.subcore_barrier()

    def chunk_body(c, carry):
      base = wid * epw + c * CH
      pltpu.sync_copy(src_h.at[pl.ds(base, CH)], src_v)
      pltpu.sync_copy(eid_h.at[pl.ds(base, CH)], eid_v)
      pltpu.sync_copy(dst_h.at[pl.ds(base, CH)], dst_v)
      pltpu.sync_copy(norm_h.at[pl.ds(base, CH)], norm_v)
      for g in range(CH // LANES):
        sl = pl.ds(g * LANES, LANES)
        idx_v[sl] = eid_v[sl] * n_nodes + src_v[sl]
      pltpu.async_copy(table_h.at[idx_v], rows_v, sem).wait()

      def grp_body(g, carry2):
        e0 = g * LANES
        nv = norm_v[pl.ds(e0, LANES)]
        for i in range(LANES):
          bc = nv[i]
          for j in range(d // LANES):
            sl = pl.ds(j * LANES, LANES)
            rows_v[e0 + i, sl] = rows_v[e0 + i, sl] * bc
        return carry2

      lax.fori_loop(0, CH // LANES, grp_body, 0)

      pltpu.sync_copy(rows_v, acc.at[dst_v], add=True)
      return carry

    lax.fori_loop(0, n_chunks, chunk_body, 0)
    plsc.subcore_barrier()

    row0 = sid * rows_per_sub
    pltpu.sync_copy(acc.at[pl.ds(row0, rows_per_sub)],
                    out_h.at[pl.ds(cid * n_nodes + row0, rows_per_sub)])
    if tail_rows:
      @pl.when(sid == 0)
      def _():
        pltpu.sync_copy(
            acc.at[pl.ds(NS * rows_per_sub, tail_rows)],
            out_h.at[pl.ds(cid * n_nodes + NS * rows_per_sub, tail_rows)])

  return k(table, src, eid, dst, norm, zeros_nd)


def kernel(node_in_feat, edge_index, edge_id, norm, W_in, b_in, rel_W, W0,
           W_out, b_out):
  n, _ = node_in_feat.shape
  e = edge_index.shape[1]
  num_layers, r, d, _ = rel_W.shape

  src = edge_index[0].astype(jnp.int32)
  dst = edge_index[1].astype(jnp.int32)
  eid = edge_id.astype(jnp.int32)
  norm32 = norm.astype(jnp.float32)

  ep = ((e + NW * CH - 1) // (NW * CH)) * (NW * CH)
  pad = ep - e
  src_p = jnp.pad(src, (0, pad))
  dst_p = jnp.pad(dst, (0, pad))
  eid_p = jnp.pad(eid, (0, pad))
  norm_p = jnp.pad(norm32, (0, pad))
  zeros_nd = jnp.zeros((n, d), jnp.float32)

  h = _mm_bias_relu(node_in_feat, W_in, b_in)
  for l in range(num_layers):
    table = _mm_rel(h, rel_W[l])
    part = _sc_edge_agg(table, src_p, eid_p, dst_p, norm_p, zeros_nd, n, d)
    h = _mm_apply(h, W0[l], part)
  return _mm_bias_relu(h, W_out, b_out)
